# baseline (speedup 1.0000x reference)
import functools

import jax
import jax.numpy as jnp
from jax import lax
from jax.experimental import pallas as pl
from jax.experimental.pallas import tpu as pltpu
from jax.experimental.pallas import tpu_sc as plsc


D_MODEL = 1024
N_WORKERS = 32
CHUNK = 16
NSX = 4
NST = 3
GROUP = 12


def _sc_body(x_hbm, pos_hbm, tab_hbm, out_hbm, idx_v, bufx_v, buft_v,
             sem_x, sem_t, sem_o):
    wid = lax.axis_index("s") * 2 + lax.axis_index("c")
    rows_per_w = x_hbm.shape[0] // N_WORKERS
    base = wid * rows_per_w
    nchunk = rows_per_w // CHUNK

    pltpu.sync_copy(pos_hbm.at[pl.ds(base, rows_per_w)], idx_v)

    def start_x(c, s):
        off = base + c * CHUNK
        pltpu.async_copy(x_hbm.at[pl.ds(off, CHUNK)], bufx_v.at[s],
                         sem_x.at[s])

    def start_t(c, s):
        pltpu.async_copy(tab_hbm.at[idx_v.at[pl.ds(c * CHUNK, CHUNK)]],
                         buft_v.at[s], sem_t.at[s])

    def wait_x(s):
        pltpu.make_async_copy(x_hbm.at[pl.ds(base, CHUNK)], bufx_v.at[s],
                              sem_x.at[s]).wait()

    def wait_t(c, s):
        pltpu.make_async_copy(tab_hbm.at[idx_v.at[pl.ds(c * CHUNK, CHUNK)]],
                              buft_v.at[s], sem_t.at[s]).wait()

    def wait_out(s):
        pltpu.make_async_copy(bufx_v.at[s], out_hbm.at[pl.ds(base, CHUNK)],
                              sem_o.at[s]).wait()

    def chunk_step(c, sx, st, first, last):
        if not first:

            @pl.when(c >= 2)
            def _():
                wait_out((sx + 2) % NSX)

        if not last:

            @pl.when(c + 2 < nchunk)
            def _():
                start_x(c + 2, (sx + 2) % NSX)
                start_t(c + 2, (st + 2) % NST)

        wait_x(sx)
        wait_t(c, st)

        def row_body(r, c2):
            for j in range(D_MODEL // 16):
                sl = pl.ds(j * 16, 16)
                plsc.addupdate(bufx_v.at[sx, r, sl], buft_v[st, r, sl])
            return c2

        lax.fori_loop(0, CHUNK, row_body, 0)
        off = base + c * CHUNK
        pltpu.async_copy(bufx_v.at[sx], out_hbm.at[pl.ds(off, CHUNK)],
                         sem_o.at[sx])

    for c in range(2):
        start_x(c, c)
        start_t(c, c)

    for c in range(2):
        chunk_step(c, c % NSX, c % NST, first=True, last=False)

    ngroups = (nchunk - 2 - 2) // GROUP

    def group_body(g, carry):
        c0 = 2 + g * GROUP
        for k in range(GROUP):
            chunk_step(c0 + k, (2 + k) % NSX, (2 + k) % NST,
                       first=False, last=False)
        return carry

    lax.fori_loop(0, ngroups, group_body, 0)

    for c in range(2 + ngroups * GROUP, nchunk):
        chunk_step(c, c % NSX, c % NST, first=False, last=(c + 2 >= nchunk))

    wait_out((nchunk - 2) % NSX)
    wait_out((nchunk - 1) % NSX)


@jax.jit
def _pos_encode(x2d, pos1d, table):
    n = x2d.shape[0]
    mesh = plsc.VectorSubcoreMesh(core_axis_name="c", subcore_axis_name="s")
    return pl.kernel(
        _sc_body,
        out_type=jax.ShapeDtypeStruct((n, D_MODEL), jnp.float32),
        mesh=mesh,
        scratch_types=[
            pltpu.VMEM((n // N_WORKERS,), jnp.int32),
            pltpu.VMEM((NSX, CHUNK, D_MODEL), jnp.float32),
            pltpu.VMEM((NST, CHUNK, D_MODEL), jnp.float32),
            pltpu.SemaphoreType.DMA((NSX,)),
            pltpu.SemaphoreType.DMA((NST,)),
            pltpu.SemaphoreType.DMA((NSX,)),
        ],
    )(x2d, pos1d, table)


def kernel(x, positions, table):
    b, l, d = x.shape
    x2d = x.reshape(b * l, d)
    pos1d = positions.reshape(-1).astype(jnp.int32)
    out = _pos_encode(x2d, pos1d, table)
    return out.reshape(b, l, d)


# speedup vs baseline: 1.6572x; 1.6572x over previous
import functools

import jax
import jax.numpy as jnp
from jax import lax
from jax.experimental import pallas as pl
from jax.experimental.pallas import tpu as pltpu
from jax.experimental.pallas import tpu_sc as plsc


D_MODEL = 1024
N_WORKERS = 32
CHUNK = 8
NSLOT = 6
DIST = 3


def _sc_body(x_hbm, pos_hbm, tab_hbm, out_hbm, idx_v, bufx_v, buft_v,
             sem_x, sem_t, sem_o):
    wid = lax.axis_index("s") * 2 + lax.axis_index("c")
    rows_per_w = x_hbm.shape[0] // N_WORKERS
    base = wid * rows_per_w
    nchunk = rows_per_w // CHUNK

    pltpu.sync_copy(pos_hbm.at[pl.ds(base, rows_per_w)], idx_v)

    def start_in(c, s):
        off = base + c * CHUNK
        pltpu.async_copy(tab_hbm.at[idx_v.at[pl.ds(c * CHUNK, CHUNK)]],
                         buft_v.at[s], sem_t.at[s])
        pltpu.async_copy(x_hbm.at[pl.ds(off, CHUNK)], bufx_v.at[s],
                         sem_x.at[s])

    def wait_in(c, s):
        pltpu.make_async_copy(x_hbm.at[pl.ds(base, CHUNK)], bufx_v.at[s],
                              sem_x.at[s]).wait()
        pltpu.make_async_copy(tab_hbm.at[idx_v.at[pl.ds(c * CHUNK, CHUNK)]],
                              buft_v.at[s], sem_t.at[s]).wait()

    def wait_out(s):
        pltpu.make_async_copy(bufx_v.at[s], out_hbm.at[pl.ds(base, CHUNK)],
                              sem_o.at[s]).wait()

    def chunk_step(c, s):
        p = (s + DIST) % NSLOT

        @pl.when(c >= NSLOT - DIST)
        def _():
            wait_out(p)

        @pl.when(c + DIST < nchunk)
        def _():
            start_in(c + DIST, p)

        wait_in(c, s)

        def row_body(r, c2):
            for j in range(D_MODEL // 16):
                sl = pl.ds(j * 16, 16)
                plsc.addupdate(bufx_v.at[s, r, sl], buft_v[s, r, sl])
            return c2

        lax.fori_loop(0, CHUNK, row_body, 0)
        off = base + c * CHUNK
        pltpu.async_copy(bufx_v.at[s], out_hbm.at[pl.ds(off, CHUNK)],
                         sem_o.at[s])

    for c in range(DIST):
        start_in(c, c % NSLOT)

    ngroups = nchunk // NSLOT

    def group_body(g, carry):
        c0 = g * NSLOT
        for k in range(NSLOT):
            chunk_step(c0 + k, k)
        return carry

    lax.fori_loop(0, ngroups, group_body, 0)

    for c in range(ngroups * NSLOT, nchunk):
        chunk_step(c, c % NSLOT)

    for c in range(nchunk - DIST, nchunk):
        wait_out(c % NSLOT)


@jax.jit
def _pos_encode(x2d, pos1d, table):
    n = x2d.shape[0]
    mesh = plsc.VectorSubcoreMesh(core_axis_name="c", subcore_axis_name="s")
    return pl.kernel(
        _sc_body,
        out_type=jax.ShapeDtypeStruct((n, D_MODEL), jnp.float32),
        mesh=mesh,
        scratch_types=[
            pltpu.VMEM((n // N_WORKERS,), jnp.int32),
            pltpu.VMEM((NSLOT, CHUNK, D_MODEL), jnp.float32),
            pltpu.VMEM((NSLOT, CHUNK, D_MODEL), jnp.float32),
            pltpu.SemaphoreType.DMA((NSLOT,)),
            pltpu.SemaphoreType.DMA((NSLOT,)),
            pltpu.SemaphoreType.DMA((NSLOT,)),
        ],
    )(x2d, pos1d, table)


def kernel(x, positions, table):
    b, l, d = x.shape
    x2d = x.reshape(b * l, d)
    pos1d = positions.reshape(-1).astype(jnp.int32)
    out = _pos_encode(x2d, pos1d, table)
    return out.reshape(b, l, d)
